# Initial kernel scaffold; baseline (speedup 1.0000x reference)
import jax
import jax.numpy as jnp
from jax.experimental import pallas as pl


def kernel(x, weight):
    raise NotImplementedError("write your pallas kernel here")


# speedup vs baseline: 1.4356x; 1.4356x over previous
import functools

import jax
import jax.numpy as jnp
from jax import lax
from jax.experimental import pallas as pl
from jax.experimental.pallas import tpu as pltpu
from jax.experimental.pallas import tpu_sc as plsc

CH = 128


@functools.lru_cache(maxsize=None)
def _make_gather(B, V, D):
    info = plsc.get_sparse_core_info()
    NC, NS = info.num_cores, info.num_subcores
    NW = NC * NS
    assert B % (NW * CH) == 0
    b_per_w = B // NW
    nch = b_per_w // CH

    mesh = plsc.VectorSubcoreMesh(core_axis_name="c", subcore_axis_name="s")

    @functools.partial(
        pl.kernel,
        mesh=mesh,
        out_type=jax.ShapeDtypeStruct((B, D), jnp.float32),
        scratch_types=[
            pltpu.VMEM((b_per_w,), jnp.int32),
            pltpu.VMEM((CH, D), jnp.float32),
            pltpu.SemaphoreType.DMA,
        ],
        compiler_params=pltpu.CompilerParams(use_tc_tiling_on_sc=False),
    )
    def gather_kernel(idx_hbm, table_hbm, out_hbm, idx_v, rows_v, sem):
        wid = lax.axis_index("s") * NC + lax.axis_index("c")
        base = wid * b_per_w
        pltpu.sync_copy(idx_hbm.at[pl.ds(base, b_per_w)], idx_v)

        def body(i, carry):
            off = i * CH
            pltpu.async_copy(
                table_hbm.at[idx_v.at[pl.ds(off, CH)]], rows_v, sem
            ).wait()
            pltpu.sync_copy(rows_v, out_hbm.at[pl.ds(base + off, CH)])
            return carry

        lax.fori_loop(0, nch, body, 0)

    return gather_kernel


def kernel(x, weight):
    batch, nf = x.shape
    V, D = weight.shape
    B = batch * nf
    idx = x.reshape(B).astype(jnp.int32)
    out = _make_gather(B, V, D)(idx, weight)
    return out.reshape(batch, nf, D)


# speedup vs baseline: 1.5519x; 1.0811x over previous
import functools

import jax
import jax.numpy as jnp
from jax import lax
from jax.experimental import pallas as pl
from jax.experimental.pallas import tpu as pltpu
from jax.experimental.pallas import tpu_sc as plsc

CH = 128
GCH = 512
NB = 2


@functools.lru_cache(maxsize=None)
def _make_gather(B, V, D):
    info = plsc.get_sparse_core_info()
    NC, NS = info.num_cores, info.num_subcores
    NW = NC * NS
    assert B % (NW * GCH * NB) == 0
    b_per_w = B // NW
    ng = b_per_w // GCH
    G = GCH // CH

    mesh = plsc.VectorSubcoreMesh(core_axis_name="c", subcore_axis_name="s")

    @functools.partial(
        pl.kernel,
        mesh=mesh,
        out_type=jax.ShapeDtypeStruct((B, D), jnp.float32),
        scratch_types=[
            pltpu.VMEM((b_per_w,), jnp.int32),
            pltpu.VMEM((NB, GCH, D), jnp.float32),
            pltpu.SemaphoreType.DMA,
            pltpu.SemaphoreType.DMA,
            pltpu.SemaphoreType.DMA,
        ],
        compiler_params=pltpu.CompilerParams(use_tc_tiling_on_sc=False),
    )
    def gather_kernel(idx_hbm, table_hbm, out_hbm, idx_v, rows, sem_g,
                      sem_o0, sem_o1):
        sem_o = [sem_o0, sem_o1]
        wid = lax.axis_index("s") * NC + lax.axis_index("c")
        base = wid * b_per_w
        pltpu.sync_copy(idx_hbm.at[pl.ds(base, b_per_w)], idx_v)

        def out_slice(g):
            return out_hbm.at[pl.ds(base + g * GCH, GCH)]

        def body(go, carry):
            for slot in range(NB):
                g = go * NB + slot

                @pl.when(go >= 1)
                def _():
                    pltpu.make_async_copy(
                        rows.at[slot], out_slice(g - NB), sem_o[slot]
                    ).wait()

                handles = [
                    pltpu.async_copy(
                        table_hbm.at[idx_v.at[pl.ds(g * GCH + j * CH, CH)]],
                        rows.at[slot].at[pl.ds(j * CH, CH)],
                        sem_g,
                    )
                    for j in range(G)
                ]
                for h in handles:
                    h.wait()
                pltpu.async_copy(rows.at[slot], out_slice(g), sem_o[slot])
            return carry

        lax.fori_loop(0, ng // NB, body, 0)

        for slot in range(NB):
            pltpu.make_async_copy(
                rows.at[slot], out_slice(ng - NB + slot), sem_o[slot]
            ).wait()

    return gather_kernel


def kernel(x, weight):
    batch, nf = x.shape
    V, D = weight.shape
    B = batch * nf
    idx = x.reshape(B).astype(jnp.int32)
    out = _make_gather(B, V, D)(idx, weight)
    return out.reshape(batch, nf, D)


# speedup vs baseline: 1.5528x; 1.0006x over previous
import functools

import jax
import jax.numpy as jnp
from jax import lax
from jax.experimental import pallas as pl
from jax.experimental.pallas import tpu as pltpu
from jax.experimental.pallas import tpu_sc as plsc

CH = 512
GCH = 512
NB = 2


@functools.lru_cache(maxsize=None)
def _make_gather(B, V, D):
    info = plsc.get_sparse_core_info()
    NC, NS = info.num_cores, info.num_subcores
    NW = NC * NS
    assert B % (NW * GCH * NB) == 0
    b_per_w = B // NW
    ng = b_per_w // GCH
    G = GCH // CH

    mesh = plsc.VectorSubcoreMesh(core_axis_name="c", subcore_axis_name="s")

    @functools.partial(
        pl.kernel,
        mesh=mesh,
        out_type=jax.ShapeDtypeStruct((B, D), jnp.float32),
        scratch_types=[
            pltpu.VMEM((b_per_w,), jnp.int32),
            pltpu.VMEM((NB, GCH, D), jnp.float32),
            pltpu.SemaphoreType.DMA,
            pltpu.SemaphoreType.DMA,
            pltpu.SemaphoreType.DMA,
        ],
        compiler_params=pltpu.CompilerParams(use_tc_tiling_on_sc=False),
    )
    def gather_kernel(idx_hbm, table_hbm, out_hbm, idx_v, rows, sem_g,
                      sem_o0, sem_o1):
        sem_o = [sem_o0, sem_o1]
        wid = lax.axis_index("s") * NC + lax.axis_index("c")
        base = wid * b_per_w
        pltpu.sync_copy(idx_hbm.at[pl.ds(base, b_per_w)], idx_v)

        def out_slice(g):
            return out_hbm.at[pl.ds(base + g * GCH, GCH)]

        def body(go, carry):
            for slot in range(NB):
                g = go * NB + slot

                @pl.when(go >= 1)
                def _():
                    pltpu.make_async_copy(
                        rows.at[slot], out_slice(g - NB), sem_o[slot]
                    ).wait()

                handles = [
                    pltpu.async_copy(
                        table_hbm.at[idx_v.at[pl.ds(g * GCH + j * CH, CH)]],
                        rows.at[slot].at[pl.ds(j * CH, CH)],
                        sem_g,
                    )
                    for j in range(G)
                ]
                for h in handles:
                    h.wait()
                pltpu.async_copy(rows.at[slot], out_slice(g), sem_o[slot])
            return carry

        lax.fori_loop(0, ng // NB, body, 0)

        for slot in range(NB):
            pltpu.make_async_copy(
                rows.at[slot], out_slice(ng - NB + slot), sem_o[slot]
            ).wait()

    return gather_kernel


def kernel(x, weight):
    batch, nf = x.shape
    V, D = weight.shape
    B = batch * nf
    idx = x.reshape(B).astype(jnp.int32)
    out = _make_gather(B, V, D)(idx, weight)
    return out.reshape(batch, nf, D)
